# baseline (speedup 1.0000x reference)
import functools

import jax
import jax.numpy as jnp
from jax import lax
from jax.experimental import pallas as pl
from jax.experimental.pallas import tpu as pltpu
from jax.experimental.pallas import tpu_sc as plsc

B, T, C = 4, 8192, 1024
NC, NS = 2, 16
NW = NC * NS
T_W = T // NW
R = 16
RC = R * C
NCHUNK = T_W // R
NC2 = NCHUNK // 2
LANES = 16

_mesh = plsc.VectorSubcoreMesh(core_axis_name="c", subcore_axis_name="s")


@functools.partial(
    pl.kernel,
    out_type=jax.ShapeDtypeStruct((B, T * C), jnp.float32),
    mesh=_mesh,
    scratch_types=[
        pltpu.VMEM((RC,), jnp.float32),
        pltpu.VMEM((RC,), jnp.float32),
        pltpu.VMEM((RC,), jnp.float32),
        pltpu.VMEM((RC,), jnp.float32),
        pltpu.SemaphoreType.DMA,
        pltpu.SemaphoreType.DMA,
        pltpu.SemaphoreType.DMA,
        pltpu.SemaphoreType.DMA,
        pltpu.SemaphoreType.DMA,
        pltpu.SemaphoreType.DMA,
    ],
)
def _pe_add_sc(x_hbm, pe_hbm, out_hbm, xb0, xb1, pb0, pb1,
               si0, si1, so0, so1, sp0, sp1):
    xbufs = (xb0, xb1)
    pbufs = (pb0, pb1)
    sin = (si0, si1)
    sout = (so0, so1)
    spe = (sp0, sp1)

    wid = lax.axis_index("s") * NC + lax.axis_index("c")
    e0_w = wid * (T_W * C)

    def e0_of(chunk):
        return e0_w + chunk * RC

    def start_in(b, chunk, p):
        pltpu.async_copy(x_hbm.at[b, pl.ds(e0_of(chunk), RC)], xbufs[p],
                         sin[p])

    def wait_in(b, chunk, p):
        pltpu.make_async_copy(
            x_hbm.at[b, pl.ds(e0_of(chunk), RC)], xbufs[p], sin[p]).wait()

    def start_out(b, chunk, p):
        pltpu.async_copy(xbufs[p], out_hbm.at[b, pl.ds(e0_of(chunk), RC)],
                         sout[p])

    def wait_out(b, chunk, p):
        pltpu.make_async_copy(
            xbufs[p], out_hbm.at[b, pl.ds(e0_of(chunk), RC)], sout[p]).wait()

    def start_pe(chunk, q):
        pltpu.async_copy(pe_hbm.at[pl.ds(e0_of(chunk), RC)], pbufs[q], spe[q])

    def wait_pe(chunk, q):
        pltpu.make_async_copy(
            pe_hbm.at[pl.ds(e0_of(chunk), RC)], pbufs[q], spe[q]).wait()

    def add_pe(p, q):
        xb, pb = xbufs[p], pbufs[q]

        @plsc.parallel_loop(0, RC, LANES, unroll=8)
        def _(i):
            plsc.addupdate(xb.at[pl.ds(i, LANES)], pb[pl.ds(i, LANES)])

    start_pe(0, 0)
    start_in(0, 0, 0)

    def outer(c2, carry):
        for cc in range(2):
            chunk = c2 * 2 + cc
            q = cc
            if cc == 0:
                start_pe(chunk + 1, 1)
            else:
                @pl.when(c2 < NC2 - 1)
                def _():
                    start_pe(chunk + 1, 0)
            wait_pe(chunk, q)
            for b in range(B):
                p = b % 2
                o = 1 - p
                wait_in(b, chunk, p)
                add_pe(p, q)
                start_out(b, chunk, p)
                if b < B - 1:
                    if b >= 1:
                        wait_out(b - 1, chunk, o)
                        start_in(b + 1, chunk, o)
                    elif cc == 1:
                        wait_out(B - 1, chunk - 1, o)
                        start_in(b + 1, chunk, o)
                    else:
                        @pl.when(c2 >= 1)
                        def _():
                            wait_out(B - 1, chunk - 1, o)
                        start_in(b + 1, chunk, o)
                else:
                    if cc == 0:
                        wait_out(B - 2, chunk, o)
                        start_in(0, chunk + 1, o)
                    else:
                        @pl.when(c2 < NC2 - 1)
                        def _():
                            wait_out(B - 2, chunk, o)
                            start_in(0, chunk + 1, o)
        return carry

    lax.fori_loop(0, NC2, outer, 0)

    wait_out(B - 2, NCHUNK - 1, 0)
    wait_out(B - 1, NCHUNK - 1, 1)


def kernel(x, pos_emb):
    out = _pe_add_sc(x.reshape(B, T * C), pos_emb.reshape(T * C))
    return out.reshape(B, T, C)


# speedup vs baseline: 1.7386x; 1.7386x over previous
import functools

import jax
import jax.numpy as jnp
from jax import lax
from jax.experimental import pallas as pl
from jax.experimental.pallas import tpu as pltpu
from jax.experimental.pallas import tpu_sc as plsc

B, T, C = 4, 8192, 1024
NC, NS = 2, 16
NW = NC * NS
T_W = T // NW
R = 16
NCHUNK = T_W // R
NC2 = NCHUNK // 2
LANES = 16
NVEC = C // LANES

_mesh = plsc.VectorSubcoreMesh(core_axis_name="c", subcore_axis_name="s")


@functools.partial(
    pl.kernel,
    out_type=jax.ShapeDtypeStruct((B, T, C), jnp.float32),
    mesh=_mesh,
    scratch_types=[
        pltpu.VMEM((R, C), jnp.float32),
        pltpu.VMEM((R, C), jnp.float32),
        pltpu.VMEM((R, C), jnp.float32),
        pltpu.VMEM((R, C), jnp.float32),
        pltpu.SemaphoreType.DMA,
        pltpu.SemaphoreType.DMA,
        pltpu.SemaphoreType.DMA,
        pltpu.SemaphoreType.DMA,
        pltpu.SemaphoreType.DMA,
        pltpu.SemaphoreType.DMA,
    ],
)
def _pe_add_sc(x_hbm, pe_hbm, out_hbm, xb0, xb1, pb0, pb1,
               si0, si1, so0, so1, sp0, sp1):
    xbufs = (xb0, xb1)
    pbufs = (pb0, pb1)
    sin = (si0, si1)
    sout = (so0, so1)
    spe = (sp0, sp1)

    wid = lax.axis_index("s") * NC + lax.axis_index("c")
    tw0 = wid * T_W

    def t0_of(chunk):
        return tw0 + chunk * R

    def start_in(b, chunk, p):
        pltpu.async_copy(x_hbm.at[b, pl.ds(t0_of(chunk), R)], xbufs[p], sin[p])

    def wait_in(b, chunk, p):
        pltpu.make_async_copy(
            x_hbm.at[b, pl.ds(t0_of(chunk), R)], xbufs[p], sin[p]).wait()

    def start_out(b, chunk, p):
        pltpu.async_copy(xbufs[p], out_hbm.at[b, pl.ds(t0_of(chunk), R)],
                         sout[p])

    def wait_out(b, chunk, p):
        pltpu.make_async_copy(
            xbufs[p], out_hbm.at[b, pl.ds(t0_of(chunk), R)], sout[p]).wait()

    def start_pe(chunk, q):
        pltpu.async_copy(pe_hbm.at[pl.ds(t0_of(chunk), R)], pbufs[q], spe[q])

    def wait_pe(chunk, q):
        pltpu.make_async_copy(
            pe_hbm.at[pl.ds(t0_of(chunk), R)], pbufs[q], spe[q]).wait()

    def add_pe(p, q):
        xb, pb = xbufs[p], pbufs[q]

        def row(r, carry):
            for j in range(NVEC):
                sl = pl.ds(j * LANES, LANES)
                xb[r, sl] = xb[r, sl] + pb[r, sl]
            return carry

        lax.fori_loop(0, R, row, 0)

    start_pe(0, 0)
    start_in(0, 0, 0)

    def outer(c2, carry):
        for cc in range(2):
            chunk = c2 * 2 + cc
            q = cc
            if cc == 0:
                start_pe(chunk + 1, 1)
            else:
                @pl.when(c2 < NC2 - 1)
                def _():
                    start_pe(chunk + 1, 0)
            wait_pe(chunk, q)
            for b in range(B):
                p = b % 2
                o = 1 - p
                wait_in(b, chunk, p)
                add_pe(p, q)
                start_out(b, chunk, p)
                if b < B - 1:
                    if b >= 1:
                        wait_out(b - 1, chunk, o)
                        start_in(b + 1, chunk, o)
                    elif cc == 1:
                        wait_out(B - 1, chunk - 1, o)
                        start_in(b + 1, chunk, o)
                    else:
                        @pl.when(c2 >= 1)
                        def _():
                            wait_out(B - 1, chunk - 1, o)
                        start_in(b + 1, chunk, o)
                else:
                    if cc == 0:
                        wait_out(B - 2, chunk, o)
                        start_in(0, chunk + 1, o)
                    else:
                        @pl.when(c2 < NC2 - 1)
                        def _():
                            wait_out(B - 2, chunk, o)
                            start_in(0, chunk + 1, o)
        return carry

    lax.fori_loop(0, NC2, outer, 0)

    wait_out(B - 2, NCHUNK - 1, 0)
    wait_out(B - 1, NCHUNK - 1, 1)


def kernel(x, pos_emb):
    return _pe_add_sc(x, pos_emb)


# speedup vs baseline: 2.9118x; 1.6748x over previous
import functools

import jax
import jax.numpy as jnp
from jax import lax
from jax.experimental import pallas as pl
from jax.experimental.pallas import tpu as pltpu
from jax.experimental.pallas import tpu_sc as plsc

B, T, C = 4, 8192, 1024
NC, NS = 2, 16
NW = NC * NS
T_W = T // NW
R = 16
NCHUNK = T_W // R
NC2 = NCHUNK // 2
LANES = 16
NVEC = C // LANES

_mesh = plsc.VectorSubcoreMesh(core_axis_name="c", subcore_axis_name="s")


@functools.partial(
    pl.kernel,
    out_type=jax.ShapeDtypeStruct((B, T, C), jnp.float32),
    mesh=_mesh,
    scratch_types=[
        pltpu.VMEM((R, C), jnp.float32),
        pltpu.VMEM((R, C), jnp.float32),
        pltpu.VMEM((R, C), jnp.float32),
        pltpu.VMEM((R, C), jnp.float32),
        pltpu.VMEM((R, C), jnp.float32),
        pltpu.VMEM((R, C), jnp.float32),
        pltpu.SemaphoreType.DMA,
        pltpu.SemaphoreType.DMA,
        pltpu.SemaphoreType.DMA,
        pltpu.SemaphoreType.DMA,
        pltpu.SemaphoreType.DMA,
        pltpu.SemaphoreType.DMA,
        pltpu.SemaphoreType.DMA,
        pltpu.SemaphoreType.DMA,
        pltpu.SemaphoreType.DMA,
        pltpu.SemaphoreType.DMA,
    ],
)
def _pe_add_sc(x_hbm, pe_hbm, out_hbm,
               xb0, xb1, xb2, xb3, pb0, pb1,
               si0, si1, si2, si3, so0, so1, so2, so3, sp0, sp1):
    xbufs = (xb0, xb1, xb2, xb3)
    pbufs = (pb0, pb1)
    sin = (si0, si1, si2, si3)
    sout = (so0, so1, so2, so3)
    spe = (sp0, sp1)

    wid = lax.axis_index("s") * NC + lax.axis_index("c")
    tw0 = wid * T_W

    def t0_of(chunk):
        return tw0 + chunk * R

    def start_in(b, chunk):
        pltpu.async_copy(x_hbm.at[b, pl.ds(t0_of(chunk), R)], xbufs[b],
                         sin[b])

    def wait_in(b, chunk):
        pltpu.make_async_copy(
            x_hbm.at[b, pl.ds(t0_of(chunk), R)], xbufs[b], sin[b]).wait()

    def start_out(b, chunk):
        pltpu.async_copy(xbufs[b], out_hbm.at[b, pl.ds(t0_of(chunk), R)],
                         sout[b])

    def wait_out(b, chunk):
        pltpu.make_async_copy(
            xbufs[b], out_hbm.at[b, pl.ds(t0_of(chunk), R)], sout[b]).wait()

    def start_pe(chunk, q):
        pltpu.async_copy(pe_hbm.at[pl.ds(t0_of(chunk), R)], pbufs[q], spe[q])

    def wait_pe(chunk, q):
        pltpu.make_async_copy(
            pe_hbm.at[pl.ds(t0_of(chunk), R)], pbufs[q], spe[q]).wait()

    def add_pe(b, q):
        xb, pb = xbufs[b], pbufs[q]

        def row(r, carry):
            for j in range(NVEC):
                sl = pl.ds(j * LANES, LANES)
                xb[r, sl] = xb[r, sl] + pb[r, sl]
            return carry

        lax.fori_loop(0, R, row, 0)

    start_pe(0, 0)
    start_in(0, 0)
    start_in(1, 0)

    def outer(c2, carry):
        for cc in range(2):
            chunk = c2 * 2 + cc
            q = cc
            if cc == 0:
                start_pe(chunk + 1, 1)
            else:
                @pl.when(c2 < NC2 - 1)
                def _():
                    start_pe(chunk + 1, 0)
            wait_pe(chunk, q)
            for b in range(B):
                sub = cc * 4 + b
                wait_in(b, chunk)
                add_pe(b, q)
                start_out(b, chunk)
                nb = (b + 2) % B
                nchunk = chunk + (b + 2) // B
                if sub <= 5:
                    if sub >= 2:
                        wait_out(nb, nchunk - 1)
                        start_in(nb, nchunk)
                    else:
                        @pl.when(c2 >= 1)
                        def _():
                            wait_out(nb, nchunk - 1)
                        start_in(nb, nchunk)
                else:
                    @pl.when(c2 < NC2 - 1)
                    def _():
                        wait_out(nb, nchunk - 1)
                        start_in(nb, nchunk)
        return carry

    lax.fori_loop(0, NC2, outer, 0)

    for b in range(B):
        wait_out(b, NCHUNK - 1)


def kernel(x, pos_emb):
    return _pe_add_sc(x, pos_emb)


# speedup vs baseline: 3.1831x; 1.0932x over previous
import functools

import jax
import jax.numpy as jnp
from jax import lax
from jax.experimental import pallas as pl
from jax.experimental.pallas import tpu as pltpu
from jax.experimental.pallas import tpu_sc as plsc

B, T, C = 4, 8192, 1024
NC, NS = 2, 16
NW = NC * NS
T_W = T // NW
R = 8
NCHUNK = T_W // R
NC2 = NCHUNK // 2
LANES = 16
NVEC = C // LANES

_mesh = plsc.VectorSubcoreMesh(core_axis_name="c", subcore_axis_name="s")


@functools.partial(
    pl.kernel,
    out_type=jax.ShapeDtypeStruct((B, T, C), jnp.float32),
    mesh=_mesh,
    scratch_types=(
        [pltpu.VMEM((R, C), jnp.float32)] * 8
        + [pltpu.VMEM((R, C), jnp.float32)] * 2
        + [pltpu.SemaphoreType.DMA] * 8
        + [pltpu.SemaphoreType.DMA] * 8
        + [pltpu.SemaphoreType.DMA] * 2
    ),
)
def _pe_add_sc(x_hbm, pe_hbm, out_hbm, *refs):
    xbufs = refs[0:8]
    pbufs = refs[8:10]
    sin = refs[10:18]
    sout = refs[18:26]
    spe = refs[26:28]

    wid = lax.axis_index("s") * NC + lax.axis_index("c")
    tw0 = wid * T_W

    def t0_of(chunk):
        return tw0 + chunk * R

    def start_in(b, chunk, u):
        pltpu.async_copy(x_hbm.at[b, pl.ds(t0_of(chunk), R)], xbufs[u],
                         sin[u])

    def wait_in(b, chunk, u):
        pltpu.make_async_copy(
            x_hbm.at[b, pl.ds(t0_of(chunk), R)], xbufs[u], sin[u]).wait()

    def start_out(b, chunk, u):
        pltpu.async_copy(xbufs[u], out_hbm.at[b, pl.ds(t0_of(chunk), R)],
                         sout[u])

    def wait_out(b, chunk, u):
        pltpu.make_async_copy(
            xbufs[u], out_hbm.at[b, pl.ds(t0_of(chunk), R)], sout[u]).wait()

    def start_pe(chunk, q):
        pltpu.async_copy(pe_hbm.at[pl.ds(t0_of(chunk), R)], pbufs[q], spe[q])

    def wait_pe(chunk, q):
        pltpu.make_async_copy(
            pe_hbm.at[pl.ds(t0_of(chunk), R)], pbufs[q], spe[q]).wait()

    def add_pe(u, q):
        xb, pb = xbufs[u], pbufs[q]

        def row(r, carry):
            for j in range(NVEC):
                sl = pl.ds(j * LANES, LANES)
                xb[r, sl] = xb[r, sl] + pb[r, sl]
            return carry

        lax.fori_loop(0, R, row, 0)

    start_pe(0, 0)
    for b in range(B):
        start_in(b, 0, b)

    def outer(c2, carry):
        for cc in range(2):
            chunk = c2 * 2 + cc
            q = cc
            if cc == 0:
                start_pe(chunk + 1, 1)
            else:
                @pl.when(c2 < NC2 - 1)
                def _():
                    start_pe(chunk + 1, 0)
            wait_pe(chunk, q)
            for b in range(B):
                u = cc * 4 + b
                v = (1 - cc) * 4 + b
                wait_in(b, chunk, u)
                add_pe(u, q)
                start_out(b, chunk, u)
                if cc == 1:
                    @pl.when(c2 < NC2 - 1)
                    def _():
                        wait_out(b, chunk - 1, v)
                        start_in(b, chunk + 1, v)
                else:
                    @pl.when(c2 >= 1)
                    def _():
                        wait_out(b, chunk - 1, v)
                    start_in(b, chunk + 1, v)
        return carry

    lax.fori_loop(0, NC2, outer, 0)

    for b in range(B):
        wait_out(b, NCHUNK - 2, b)
    for b in range(B):
        wait_out(b, NCHUNK - 1, 4 + b)


def kernel(x, pos_emb):
    return _pe_add_sc(x, pos_emb)
